# Initial kernel scaffold; baseline (speedup 1.0000x reference)
import jax
import jax.numpy as jnp
from jax.experimental import pallas as pl


def kernel(x, stem_w, stem_b, L0B0_conv1_w, L0B0_conv1_b, L0B0_conv2_w, L0B0_conv2_b, L0B0_conv3_w, L0B0_conv3_b, L0B0_down_w, L0B0_down_b, L0B1_conv1_w, L0B1_conv1_b, L0B1_conv2_w, L0B1_conv2_b, L0B1_conv3_w, L0B1_conv3_b, L0B2_conv1_w, L0B2_conv1_b, L0B2_conv2_w, L0B2_conv2_b, L0B2_conv3_w, L0B2_conv3_b, L1B0_conv1_w, L1B0_conv1_b, L1B0_conv2_w, L1B0_conv2_b, L1B0_conv3_w, L1B0_conv3_b, L1B0_down_w, L1B0_down_b, L1B1_conv1_w, L1B1_conv1_b, L1B1_conv2_w, L1B1_conv2_b, L1B1_conv3_w, L1B1_conv3_b, L1B2_conv1_w, L1B2_conv1_b, L1B2_conv2_w, L1B2_conv2_b, L1B2_conv3_w, L1B2_conv3_b, L1B3_conv1_w, L1B3_conv1_b, L1B3_conv2_w, L1B3_conv2_b, L1B3_conv3_w, L1B3_conv3_b, L2B0_conv1_w, L2B0_conv1_b, L2B0_conv2_w, L2B0_conv2_b, L2B0_conv3_w, L2B0_conv3_b, L2B0_down_w, L2B0_down_b, L2B1_conv1_w, L2B1_conv1_b, L2B1_conv2_w, L2B1_conv2_b, L2B1_conv3_w, L2B1_conv3_b, L2B2_conv1_w, L2B2_conv1_b, L2B2_conv2_w, L2B2_conv2_b, L2B2_conv3_w, L2B2_conv3_b, L2B3_conv1_w, L2B3_conv1_b, L2B3_conv2_w, L2B3_conv2_b, L2B3_conv3_w, L2B3_conv3_b, L2B4_conv1_w, L2B4_conv1_b, L2B4_conv2_w, L2B4_conv2_b, L2B4_conv3_w, L2B4_conv3_b, L2B5_conv1_w, L2B5_conv1_b, L2B5_conv2_w, L2B5_conv2_b, L2B5_conv3_w, L2B5_conv3_b, L3B0_conv1_w, L3B0_conv1_b, L3B0_conv2_w, L3B0_conv2_b, L3B0_conv3_w, L3B0_conv3_b, L3B0_down_w, L3B0_down_b, L3B1_conv1_w, L3B1_conv1_b, L3B1_conv2_w, L3B1_conv2_b, L3B1_conv3_w, L3B1_conv3_b, L3B2_conv1_w, L3B2_conv1_b, L3B2_conv2_w, L3B2_conv2_b, L3B2_conv3_w, L3B2_conv3_b, fc_w, fc_b):
    raise NotImplementedError("write your pallas kernel here")


# speedup vs baseline: 7.9278x; 7.9278x over previous
import functools

import jax
import jax.numpy as jnp
from jax.experimental import pallas as pl
from jax.experimental.pallas import tpu as pltpu


def _rup(x, m):
    return (x + m - 1) // m * m


def _mm_dims(K, N):
    tn = min(256, _rup(N, 128))
    Np = _rup(N, tn)
    if K <= 512:
        tk, Kp = K, K
    else:
        tk = 512
        for d in range(512, 127, -128):
            if K % d == 0:
                tk = d
                break
        Kp = _rup(K, tk)
    return tk, tn, Kp, Np


def _pick_tm(M):
    tm = min(512, _rup(M, 8))
    Mp = _rup(M, tm)
    if Mp // tm < 2 and M >= 16:
        tm = _rup(_rup(M, 8) // 2, 8)
        Mp = _rup(M, tm)
    return tm, Mp


def _mm_epi_kernel(x_ref, w_ref, b_ref, o_ref, acc_ref, *, relu):
    @pl.when(pl.program_id(2) == 0)
    def _():
        acc_ref[...] = jnp.zeros_like(acc_ref)

    acc_ref[...] += jnp.dot(x_ref[...], w_ref[...],
                            preferred_element_type=jnp.float32)

    @pl.when(pl.program_id(2) == pl.num_programs(2) - 1)
    def _():
        y = acc_ref[...] + b_ref[...]
        if relu:
            y = jnp.maximum(y, 0.0)
        o_ref[...] = y.astype(o_ref.dtype)


def _mm_res_epi_kernel(x_ref, w_ref, b_ref, r_ref, o_ref, acc_ref):
    @pl.when(pl.program_id(2) == 0)
    def _():
        acc_ref[...] = jnp.zeros_like(acc_ref)

    acc_ref[...] += jnp.dot(x_ref[...], w_ref[...],
                            preferred_element_type=jnp.float32)

    @pl.when(pl.program_id(2) == pl.num_programs(2) - 1)
    def _():
        y = acc_ref[...] + b_ref[...] + r_ref[...].astype(jnp.float32)
        o_ref[...] = jnp.maximum(y, 0.0).astype(o_ref.dtype)


def _fused_blk_kernel(x_ref, w2_ref, b2_ref, w3_ref, b3_ref, r_ref, o_ref,
                      *, wp, tm, np2):
    base = pl.program_id(0) * tm
    acc2 = jnp.zeros((tm, np2), jnp.float32)
    for di in range(3):
        off = di * wp
        al = (off // 8) * 8
        rem = off - al
        wide = x_ref[pl.ds(base + al, tm + 16), :]
        for dj in range(3):
            xs = wide[rem + dj:rem + dj + tm, :]
            acc2 = acc2 + jnp.dot(xs, w2_ref[di * 3 + dj],
                                  preferred_element_type=jnp.float32)
    c2 = jnp.maximum(acc2 + b2_ref[...], 0.0).astype(jnp.bfloat16)
    y = jnp.dot(c2, w3_ref[...], preferred_element_type=jnp.float32)
    y = y + b3_ref[...] + r_ref[...].astype(jnp.float32)
    o_ref[...] = jnp.maximum(y, 0.0).astype(o_ref.dtype)


def _pool9_kernel(ee_ref, eo_ref, oe_ref, oo_ref, o_ref, *, ho, wo):
    v = ee_ref[0:ho, 0:wo, :]
    v = jnp.maximum(v, ee_ref[0:ho, 1:wo + 1, :])
    v = jnp.maximum(v, ee_ref[1:ho + 1, 0:wo, :])
    v = jnp.maximum(v, ee_ref[1:ho + 1, 1:wo + 1, :])
    v = jnp.maximum(v, eo_ref[0:ho, 0:wo, :])
    v = jnp.maximum(v, eo_ref[1:ho + 1, 0:wo, :])
    v = jnp.maximum(v, oe_ref[0:ho, 0:wo, :])
    v = jnp.maximum(v, oe_ref[0:ho, 1:wo + 1, :])
    v = jnp.maximum(v, oo_ref[0:ho, 0:wo, :])
    o_ref[...] = v


def _pool_fc_kernel(x_ref, w_ref, b_ref, o_ref):
    pooled = jnp.mean(x_ref[...].astype(jnp.float32), axis=0, keepdims=True)
    o_ref[...] = jnp.dot(pooled.astype(jnp.bfloat16), w_ref[...],
                         preferred_element_type=jnp.float32) + b_ref[...]


def _matmul(x, w, bias, *, relu, residual=None):
    M, K = x.shape
    N = bias.shape[0]
    tk, tn, Kp, Np = _mm_dims(K, N)
    tm, Mp = _pick_tm(M)

    xp = x.astype(jnp.bfloat16)
    if (Mp, Kp) != (M, K):
        xp = jnp.pad(xp, ((0, Mp - M), (0, Kp - K)))
    bp = bias.reshape(1, N).astype(jnp.float32)
    if Np != N:
        bp = jnp.pad(bp, ((0, 0), (0, Np - N)))

    ins = [xp, w, bp]
    specs = [
        pl.BlockSpec((tm, tk), lambda i, j, k: (i, k)),
        pl.BlockSpec((tk, tn), lambda i, j, k: (k, j)),
        pl.BlockSpec((1, tn), lambda i, j, k: (0, j)),
    ]
    if residual is not None:
        rp = residual.astype(jnp.bfloat16)
        if rp.shape != (Mp, Np):
            rp = jnp.pad(rp, ((0, Mp - rp.shape[0]), (0, Np - rp.shape[1])))
        ins.append(rp)
        specs.append(pl.BlockSpec((tm, tn), lambda i, j, k: (i, j)))
        body = _mm_res_epi_kernel
    else:
        body = functools.partial(_mm_epi_kernel, relu=relu)

    out = pl.pallas_call(
        body,
        out_shape=jax.ShapeDtypeStruct((Mp, Np), jnp.bfloat16),
        grid_spec=pltpu.PrefetchScalarGridSpec(
            num_scalar_prefetch=0,
            grid=(Mp // tm, Np // tn, Kp // tk),
            in_specs=specs,
            out_specs=pl.BlockSpec((tm, tn), lambda i, j, k: (i, j)),
            scratch_shapes=[pltpu.VMEM((tm, tn), jnp.float32)],
        ),
        compiler_params=pltpu.CompilerParams(
            dimension_semantics=("parallel", "parallel", "arbitrary")),
    )(*ins)
    if (Mp, Np) != (M, N):
        out = out[:M, :N]
    return out


def _fused_block_tail(x, w2, b2, w3, b3, identity):
    _, H, W, cin = x.shape
    ntaps, _, Np2 = w2.shape
    planes = b2.shape[0]
    Kp3, Np3 = w3.shape
    C4 = b3.shape[0]

    wp = W + 2
    M2 = H * wp
    tm = M2
    for cand in range(min(576, M2 // 2), 7, -8):
        if M2 % cand == 0:
            tm = cand
            break

    xpad = jnp.pad(x[0].astype(jnp.bfloat16), ((1, 1), (1, 1), (0, 0)))
    xflat = xpad.reshape((H + 2) * wp, cin)
    rows = _rup(M2 + 2 * wp + 2 + 24, 8)
    xflat = jnp.pad(xflat, ((0, rows - xflat.shape[0]), (0, 0)))

    b2p = b2.reshape(1, planes).astype(jnp.float32)
    if Np2 != planes:
        b2p = jnp.pad(b2p, ((0, 0), (0, Np2 - planes)))
    w3p = w3 if Kp3 == Np2 else jnp.pad(w3, ((0, Np2 - Kp3), (0, 0)))
    b3p = b3.reshape(1, C4).astype(jnp.float32)
    if Np3 != C4:
        b3p = jnp.pad(b3p, ((0, 0), (0, Np3 - C4)))

    res = jnp.pad(identity[0].astype(jnp.bfloat16), ((0, 0), (0, 2), (0, 0)))
    res = res.reshape(M2, C4)
    if Np3 != C4:
        res = jnp.pad(res, ((0, 0), (0, Np3 - C4)))

    out = pl.pallas_call(
        functools.partial(_fused_blk_kernel, wp=wp, tm=tm, np2=Np2),
        out_shape=jax.ShapeDtypeStruct((M2, Np3), jnp.bfloat16),
        grid_spec=pltpu.PrefetchScalarGridSpec(
            num_scalar_prefetch=0,
            grid=(M2 // tm,),
            in_specs=[
                pl.BlockSpec((rows, cin), lambda i: (0, 0)),
                pl.BlockSpec((ntaps, cin, Np2), lambda i: (0, 0, 0)),
                pl.BlockSpec((1, Np2), lambda i: (0, 0)),
                pl.BlockSpec((Np2, Np3), lambda i: (0, 0)),
                pl.BlockSpec((1, Np3), lambda i: (0, 0)),
                pl.BlockSpec((tm, Np3), lambda i: (i, 0)),
            ],
            out_specs=pl.BlockSpec((tm, Np3), lambda i: (i, 0)),
        ),
        compiler_params=pltpu.CompilerParams(
            dimension_semantics=("parallel",)),
    )(xflat, w2, b2p, w3p, b3p, res)

    out = out.reshape(H, wp, Np3)[:, :W, :C4]
    return out.reshape(1, H, W, C4)


def _patches(x, k, stride, pad):
    x = jnp.pad(x, ((pad, pad), (pad, pad), (0, 0)))
    H, W, _ = x.shape
    Ho = (H - k) // stride + 1
    Wo = (W - k) // stride + 1
    cols = []
    for i in range(k):
        for j in range(k):
            cols.append(x[i:i + stride * (Ho - 1) + 1:stride,
                          j:j + stride * (Wo - 1) + 1:stride, :])
    return jnp.concatenate(cols, axis=-1), Ho, Wo


def _conv1x1(x, w, b, *, stride=1, relu=True, residual=None):
    N, H, W, cin = x.shape
    if stride > 1:
        x = x[:, ::stride, ::stride, :]
    _, Ho, Wo, _ = x.shape
    r2 = residual.reshape(Ho * Wo, -1) if residual is not None else None
    y = _matmul(x.reshape(Ho * Wo, cin), w, b, relu=relu, residual=r2)
    return y.reshape(1, Ho, Wo, -1)


def _conv_s2_im2col(x, w, b, *, k, pad, relu=True):
    cin = x.shape[-1]
    p, Ho, Wo = _patches(x[0].astype(jnp.bfloat16), k, 2, pad)
    y = _matmul(p.reshape(Ho * Wo, k * k * cin), w, b, relu=relu)
    return y.reshape(1, Ho, Wo, -1)


def _stem_and_pool(x, stem_w, stem_b):
    data = x[0].T.reshape(8, 256, 3).astype(jnp.bfloat16)
    slab = jnp.pad(data, ((3, 6), (3, 3), (0, 0)))
    p, ho, wo = _patches_s2_7x7(slab)
    stem_rows = _matmul(p.reshape(6 * 128, 147), stem_w, stem_b,
                        relu=True).reshape(6, 128, 64)
    cvec = jnp.maximum(stem_b, 0.0).astype(jnp.bfloat16)

    band = jnp.concatenate(
        [stem_rows, jnp.broadcast_to(cvec, (2, 128, 64))], axis=0)
    ninf = jnp.float32(-jnp.inf).astype(jnp.bfloat16)
    pp = jnp.pad(band, ((1, 1), (1, 1), (0, 0)), constant_values=ninf)
    ee = pp[0::2, 0::2, :]
    eo = pp[0::2, 1::2, :]
    oe = pp[1::2, 0::2, :]
    oo = pp[1::2, 1::2, :]
    pool = pl.pallas_call(
        functools.partial(_pool9_kernel, ho=4, wo=64),
        out_shape=jax.ShapeDtypeStruct((4, 64, 64), jnp.bfloat16),
        in_specs=[pl.BlockSpec(memory_space=pltpu.MemorySpace.VMEM)] * 4,
        out_specs=pl.BlockSpec(memory_space=pltpu.MemorySpace.VMEM),
    )(ee, eo, oe, oo)
    full = jnp.concatenate(
        [pool, jnp.broadcast_to(cvec, (60, 64, 64))], axis=0)
    return full.reshape(1, 64, 64, 64)


def _patches_s2_7x7(slab):
    cols = []
    for i in range(7):
        for j in range(7):
            cols.append(slab[i:i + 11:2, j:j + 255:2, :])
    return jnp.concatenate(cols, axis=-1), 6, 128


def _head(x, fc_w, fc_b):
    _, H, W, C = x.shape
    ncp = fc_w.shape[1]
    nc = fc_b.shape[0]
    bp = jnp.pad(fc_b.reshape(1, nc).astype(jnp.float32),
                 ((0, 0), (0, ncp - nc)))
    out = pl.pallas_call(
        _pool_fc_kernel,
        out_shape=jax.ShapeDtypeStruct((1, ncp), jnp.float32),
        in_specs=[pl.BlockSpec(memory_space=pltpu.MemorySpace.VMEM)] * 3,
        out_specs=pl.BlockSpec(memory_space=pltpu.MemorySpace.VMEM),
    )(x.reshape(H * W, C), fc_w, bp)
    return out[:, :nc]


_NB = [3, 4, 6, 3]
_LSTRIDE = [1, 2, 2, 2]


def kernel(x, stem_w, stem_b, L0B0_conv1_w, L0B0_conv1_b, L0B0_conv2_w, L0B0_conv2_b, L0B0_conv3_w, L0B0_conv3_b, L0B0_down_w, L0B0_down_b, L0B1_conv1_w, L0B1_conv1_b, L0B1_conv2_w, L0B1_conv2_b, L0B1_conv3_w, L0B1_conv3_b, L0B2_conv1_w, L0B2_conv1_b, L0B2_conv2_w, L0B2_conv2_b, L0B2_conv3_w, L0B2_conv3_b, L1B0_conv1_w, L1B0_conv1_b, L1B0_conv2_w, L1B0_conv2_b, L1B0_conv3_w, L1B0_conv3_b, L1B0_down_w, L1B0_down_b, L1B1_conv1_w, L1B1_conv1_b, L1B1_conv2_w, L1B1_conv2_b, L1B1_conv3_w, L1B1_conv3_b, L1B2_conv1_w, L1B2_conv1_b, L1B2_conv2_w, L1B2_conv2_b, L1B2_conv3_w, L1B2_conv3_b, L1B3_conv1_w, L1B3_conv1_b, L1B3_conv2_w, L1B3_conv2_b, L1B3_conv3_w, L1B3_conv3_b, L2B0_conv1_w, L2B0_conv1_b, L2B0_conv2_w, L2B0_conv2_b, L2B0_conv3_w, L2B0_conv3_b, L2B0_down_w, L2B0_down_b, L2B1_conv1_w, L2B1_conv1_b, L2B1_conv2_w, L2B1_conv2_b, L2B1_conv3_w, L2B1_conv3_b, L2B2_conv1_w, L2B2_conv1_b, L2B2_conv2_w, L2B2_conv2_b, L2B2_conv3_w, L2B2_conv3_b, L2B3_conv1_w, L2B3_conv1_b, L2B3_conv2_w, L2B3_conv2_b, L2B3_conv3_w, L2B3_conv3_b, L2B4_conv1_w, L2B4_conv1_b, L2B4_conv2_w, L2B4_conv2_b, L2B4_conv3_w, L2B4_conv3_b, L2B5_conv1_w, L2B5_conv1_b, L2B5_conv2_w, L2B5_conv2_b, L2B5_conv3_w, L2B5_conv3_b, L3B0_conv1_w, L3B0_conv1_b, L3B0_conv2_w, L3B0_conv2_b, L3B0_conv3_w, L3B0_conv3_b, L3B0_down_w, L3B0_down_b, L3B1_conv1_w, L3B1_conv1_b, L3B1_conv2_w, L3B1_conv2_b, L3B1_conv3_w, L3B1_conv3_b, L3B2_conv1_w, L3B2_conv1_b, L3B2_conv2_w, L3B2_conv2_b, L3B2_conv3_w, L3B2_conv3_b, fc_w, fc_b):
    P = dict(locals())

    out = _stem_and_pool(x, stem_w, stem_b)
    for li in range(4):
        for bi in range(_NB[li]):
            s = _LSTRIDE[li] if bi == 0 else 1
            w1 = P[f'L{li}B{bi}_conv1_w']
            b1 = P[f'L{li}B{bi}_conv1_b']
            w2 = P[f'L{li}B{bi}_conv2_w']
            b2 = P[f'L{li}B{bi}_conv2_b']
            w3 = P[f'L{li}B{bi}_conv3_w']
            b3 = P[f'L{li}B{bi}_conv3_b']
            if bi == 0:
                identity = _conv1x1(out, P[f'L{li}B{bi}_down_w'],
                                    P[f'L{li}B{bi}_down_b'],
                                    stride=s, relu=False)
            else:
                identity = out
            h = _conv1x1(out, w1, b1, relu=True)
            if s == 1:
                out = _fused_block_tail(h, w2, b2, w3, b3, identity)
            else:
                h = _conv_s2_im2col(h, w2, b2, k=3, pad=1, relu=True)
                out = _conv1x1(h, w3, b3, relu=True, residual=identity)
    return _head(out, fc_w, fc_b)
